# baseline (speedup 1.0000x reference)
import functools

import numpy as np

import jax
import jax.numpy as jnp
from jax import lax
from jax.experimental import pallas as pl
from jax.experimental.pallas import tpu as pltpu
from jax.experimental.pallas import tpu_sc as plsc

VOCAB = 100000
DIM = 128
BATCH = 4096
SEQ = 50

NC = 2
NS = 16
NW = NC * NS
B_PER_W = BATCH // NW
CHUNK = 8
ROWS = CHUNK * SEQ
NCHUNK = B_PER_W // CHUNK
DEPTH = 2
SC_ROWS = BATCH // NC

PARTS = ((0, 128), (128, 128), (256, 128), (384, 16))
NPART = len(PARTS)


def _seg_indices():
    local = np.arange(ROWS) // SEQ
    a = np.empty((NCHUNK * 3, 128), np.int32)
    b = np.empty((NCHUNK, 16), np.int32)
    for c in range(NCHUNK):
        seg = c * CHUNK + local
        for j in range(3):
            a[3 * c + j] = seg[128 * j:128 * (j + 1)]
        b[c] = seg[384:400]
    return jnp.asarray(a), jnp.asarray(b)


def _sc_gather_sum(idx_flat, table):
    mesh = plsc.VectorSubcoreMesh(core_axis_name="c", subcore_axis_name="s")
    idx_a, idx_b = _seg_indices()

    @functools.partial(
        pl.kernel,
        mesh=mesh,
        out_type=jax.ShapeDtypeStruct((BATCH, DIM), jnp.float32),
        scratch_types=[
            pltpu.VMEM((B_PER_W * SEQ,), jnp.int32),
            pltpu.VMEM((NCHUNK * 3, 128), jnp.int32),
            pltpu.VMEM((NCHUNK, 16), jnp.int32),
            *[pltpu.VMEM((ROWS, DIM), jnp.float32) for _ in range(DEPTH)],
            pltpu.VMEM_SHARED((B_PER_W, DIM), jnp.float32),
            *[pltpu.SemaphoreType.DMA for _ in range(2 * DEPTH)],
        ],
    )
    def k(idx_hbm, table_hbm, ia_hbm, ib_hbm, out_hbm,
          idx_v, ia_v, ib_v, *rest):
        bufs = rest[:DEPTH]
        shared = rest[DEPTH]
        gsems = rest[DEPTH + 1:DEPTH + 1 + DEPTH]
        ssems = rest[DEPTH + 1 + DEPTH:DEPTH + 1 + 2 * DEPTH]

        sid = lax.axis_index("s")
        wid = sid * NC + lax.axis_index("c")
        ibase = wid * (B_PER_W * SEQ)

        pltpu.sync_copy(idx_hbm.at[pl.ds(ibase, B_PER_W * SEQ)], idx_v)

        def issue(c, buf, sem):
            off = pl.multiple_of(c * ROWS, 8)
            for lo, sz in PARTS:
                pltpu.async_copy(
                    table_hbm.at[idx_v.at[pl.ds(off + lo, sz)]],
                    buf.at[pl.ds(lo, sz)], sem)

        def drain_gather(buf, sem):
            for lo, sz in PARTS:
                pltpu.make_async_copy(
                    table_hbm.at[idx_v.at[pl.ds(lo, sz)]],
                    buf.at[pl.ds(lo, sz)], sem).wait()

        issue(0, bufs[0], gsems[0])

        pltpu.sync_copy(ia_hbm, ia_v)
        pltpu.sync_copy(ib_hbm, ib_v)

        def zero_body(r, _):
            for v in range(DIM // 16):
                bufs[1][r, pl.ds(16 * v, 16)] = jnp.zeros((16,), jnp.float32)
            return 0
        lax.fori_loop(0, B_PER_W, zero_body, 0)
        pltpu.sync_copy(bufs[1].at[pl.ds(0, B_PER_W)], shared)

        def scatter(c, buf, sem):
            for p, (lo, sz) in enumerate(PARTS):
                if sz == 128:
                    irow = ia_v.at[3 * c + p]
                else:
                    irow = ib_v.at[c]
                pltpu.async_copy(
                    buf.at[pl.ds(lo, sz)],
                    shared.at[irow], sem, add=True)

        def drain_scatter(buf, sem):
            for p, (lo, sz) in enumerate(PARTS):
                if sz == 128:
                    irow = ia_v.at[0]
                else:
                    irow = ib_v.at[0]
                pltpu.make_async_copy(
                    buf.at[pl.ds(lo, sz)],
                    shared.at[irow], sem).wait()

        def ring_body(i, carry):
            for j in range(DEPTH):
                c = i * DEPTH + j
                nj = (j + 1) % DEPTH

                @pl.when(c >= 1)
                def _(nj=nj):
                    drain_scatter(bufs[nj], ssems[nj])

                @pl.when(c + 1 < NCHUNK)
                def _(c=c, nj=nj):
                    issue(c + 1, bufs[nj], gsems[nj])

                drain_gather(bufs[j], gsems[j])
                scatter(c, bufs[j], ssems[j])
            return carry

        lax.fori_loop(0, NCHUNK // DEPTH, ring_body, 0)
        lj = (NCHUNK - 1) % DEPTH
        drain_scatter(bufs[lj], ssems[lj])

        obase = pl.multiple_of(wid * B_PER_W, 8)
        pltpu.sync_copy(shared, out_hbm.at[pl.ds(obase, B_PER_W)])

    return k(idx_flat, table, idx_a, idx_b)


def _mlp_body(s_ref, w1_ref, b1_ref, w2_ref, b2_ref, w3_ref, b3_ref, o_ref):
    dn = (((1,), (1,)), ((), ()))
    x = s_ref[...] * (1.0 / SEQ)
    h = jnp.tanh(lax.dot_general(x, w1_ref[...], dn,
                                 preferred_element_type=jnp.float32)
                 + b1_ref[...])
    h = jnp.tanh(lax.dot_general(h, w2_ref[...], dn,
                                 preferred_element_type=jnp.float32)
                 + b2_ref[...])
    z = jnp.tanh(lax.dot_general(h, w3_ref[...], dn,
                                 preferred_element_type=jnp.float32)
                 + b3_ref[...])
    a = z[:, 0:1]
    b = z[:, 1:2]
    lse = jnp.logaddexp(a, b)
    o_ref[...] = jnp.concatenate([a - lse, b - lse], axis=1)


def _tc_mlp(sums, w1, b1, w2, b2, w3, b3):
    blk = 2048
    grid = BATCH // blk
    return pl.pallas_call(
        _mlp_body,
        grid=(grid,),
        in_specs=[
            pl.BlockSpec((blk, DIM), lambda i: (i, 0)),
            pl.BlockSpec(w1.shape, lambda i: (0, 0)),
            pl.BlockSpec(b1.shape, lambda i: (0, 0)),
            pl.BlockSpec(w2.shape, lambda i: (0, 0)),
            pl.BlockSpec(b2.shape, lambda i: (0, 0)),
            pl.BlockSpec(w3.shape, lambda i: (0, 0)),
            pl.BlockSpec(b3.shape, lambda i: (0, 0)),
        ],
        out_specs=pl.BlockSpec((blk, 2), lambda i: (i, 0)),
        out_shape=jax.ShapeDtypeStruct((BATCH, 2), jnp.float32),
    )(sums, w1, b1, w2, b2, w3, b3)


def kernel(input, emb_weight, out_w, out_b, out1_w, out1_b, out2_w, out2_b):
    sums = _sc_gather_sum(input.reshape(-1), emb_weight)
    return _tc_mlp(sums,
                   out_w, out_b.reshape(1, -1),
                   out1_w, out1_b.reshape(1, -1),
                   out2_w, out2_b.reshape(1, -1))
